# baseline (speedup 1.0000x reference)
import functools

import jax
import jax.numpy as jnp
from jax import lax
from jax.experimental import pallas as pl
from jax.experimental.pallas import tpu as pltpu
from jax.experimental.pallas import tpu_sc as plsc

N = 8192
K1 = 17
NC, NS, L = 2, 16, 16
NW = NC * NS
RPW = N // NW
HR = RPW // 2
HE = K1 * HR
E = K1 * RPW
CH = 128


def _sc_row_sums(data_phys, idx_t):
    mesh = plsc.VectorSubcoreMesh(
        core_axis_name="c", subcore_axis_name="s",
        num_cores=NC, num_subcores=NS)

    @functools.partial(
        pl.kernel,
        out_type=jax.ShapeDtypeStruct((N,), jnp.float32),
        mesh=mesh,
        compiler_params=pltpu.CompilerParams(needs_layout_passes=False),
        scratch_types=[
            pltpu.VMEM((K1, RPW), jnp.int32),
            pltpu.VMEM((RPW,), jnp.int32),
            pltpu.VMEM((E,), jnp.int32),
            pltpu.VMEM((E,), jnp.float32),
            pltpu.VMEM((RPW,), jnp.float32),
            pltpu.SemaphoreType.DMA,
        ],
    )
    def rowsum_kernel(data_hbm, idx_hbm, out_hbm, cols_v, ioff_v, flat_v,
                      gath_v, sums_v, sem):
        wid = lax.axis_index("s") * NC + lax.axis_index("c")
        base = wid * RPW

        pltpu.sync_copy(idx_hbm.at[:, pl.ds(base, RPW)], cols_v)

        def pre_chunk(c, carry):
            i = base + c * L + lax.iota(jnp.int32, L)
            ioff_v[pl.ds(c * L, L)] = ((i >> 3) << 16) + ((i & 7) << 7)
            return carry

        lax.fori_loop(0, RPW // L, pre_chunk, 0)

        def fire_j(h, j):
            def build_chunk(c, carry):
                r = h * HR + c * L
                col = cols_v[j, pl.ds(r, L)]
                flat_v[pl.ds(h * HE + j * HR + c * L, L)] = (
                    ioff_v[pl.ds(r, L)] + ((col >> 7) << 10) + (col & 127))
                return carry

            lax.fori_loop(0, HR // L, build_chunk, 0)
            return pltpu.async_copy(
                data_hbm.at[flat_v.at[pl.ds(h * HE + j * CH, CH)]],
                gath_v.at[pl.ds(h * HE + j * CH, CH)], sem)

        def reduce_half(h):
            def reduce_chunk(c, carry):
                acc = gath_v[pl.ds(h * HE + c * L, L)]
                for j in range(1, K1):
                    acc = acc + gath_v[pl.ds(h * HE + j * HR + c * L, L)]
                sums_v[pl.ds(h * HR + c * L, L)] = acc
                return carry

            lax.fori_loop(0, HR // L, reduce_chunk, 0)

        descs0 = [fire_j(0, j) for j in range(K1)]
        descs1 = [fire_j(1, j) for j in range(K1)]
        for d in descs0:
            d.wait()
        reduce_half(0)
        for d in descs1:
            d.wait()
        reduce_half(1)

        pltpu.sync_copy(sums_v, out_hbm.at[pl.ds(base, RPW)])

    return rowsum_kernel(data_phys, idx_t)


def _tc_softmax(a_raw):
    def body(x_ref, alpha_ref):
        x = x_ref[...]
        m = jnp.max(x)
        e = jnp.exp(x - m)
        alpha_ref[...] = e / jnp.sum(e)

    return pl.pallas_call(
        body,
        out_shape=jax.ShapeDtypeStruct((N,), jnp.float32),
    )(a_raw)


def kernel(data_input, neighbor_indices):
    idx = neighbor_indices[:, :K1].astype(jnp.int32)
    idx_t = idx.T.reshape(K1, N)
    data_phys = (data_input
                 .reshape(N // 8, 8, N // 128, 128)
                 .transpose(0, 2, 1, 3)
                 .reshape(N * N))
    a_raw = _sc_row_sums(data_phys, idx_t)
    alpha = _tc_softmax(a_raw)
    return (alpha, a_raw)


# speedup vs baseline: 1.0035x; 1.0035x over previous
import functools

import jax
import jax.numpy as jnp
from jax import lax
from jax.experimental import pallas as pl
from jax.experimental.pallas import tpu as pltpu
from jax.experimental.pallas import tpu_sc as plsc

N = 8192
K1 = 17
NC, NS, L = 2, 16, 16
NW = NC * NS
RPW = N // NW
HR = RPW // 2
HE = K1 * HR
E = K1 * RPW
CH = 128


def _sc_row_sums(data_phys, idx_t):
    mesh = plsc.VectorSubcoreMesh(
        core_axis_name="c", subcore_axis_name="s",
        num_cores=NC, num_subcores=NS)

    @functools.partial(
        pl.kernel,
        out_type=jax.ShapeDtypeStruct((N,), jnp.float32),
        mesh=mesh,
        compiler_params=pltpu.CompilerParams(needs_layout_passes=False),
        scratch_types=[
            pltpu.VMEM((K1, RPW), jnp.int32),
            pltpu.VMEM((RPW,), jnp.int32),
            pltpu.VMEM((E,), jnp.int32),
            pltpu.VMEM((E,), jnp.float32),
            pltpu.VMEM((RPW,), jnp.float32),
            pltpu.SemaphoreType.DMA,
            pltpu.SemaphoreType.DMA,
        ],
    )
    def rowsum_kernel(data_hbm, idx_hbm, out_hbm, cols_v, ioff_v, flat_v,
                      gath_v, sums_v, sem, csem):
        wid = lax.axis_index("s") * NC + lax.axis_index("c")
        base = wid * RPW

        cols_dma = pltpu.async_copy(
            idx_hbm.at[:, pl.ds(base, RPW)], cols_v, csem)

        def pre_chunk(c, carry):
            i = base + c * L + lax.iota(jnp.int32, L)
            ioff_v[pl.ds(c * L, L)] = ((i >> 3) << 16) + ((i & 7) << 7)
            return carry

        lax.fori_loop(0, RPW // L, pre_chunk, 0)
        cols_dma.wait()

        def build_half(h):
            def build_chunk(c, carry):
                r = h * HR + c * L
                ioff = ioff_v[pl.ds(r, L)]

                def build_j(j, carry):
                    col = cols_v[j, pl.ds(r, L)]
                    flat_v[pl.ds(h * HE + j * HR + c * L, L)] = (
                        ioff + ((col >> 7) << 10) + (col & 127))
                    return carry

                return lax.fori_loop(0, K1, build_j, carry)

            lax.fori_loop(0, HR // L, build_chunk, 0)

        def fire_half(h):
            return [
                pltpu.async_copy(
                    data_hbm.at[flat_v.at[pl.ds(h * HE + j * CH, CH)]],
                    gath_v.at[pl.ds(h * HE + j * CH, CH)], sem)
                for j in range(K1)
            ]

        def reduce_half(h):
            def reduce_chunk(c, carry):
                def add_j(j, acc):
                    return acc + gath_v[pl.ds(h * HE + j * HR + c * L, L)]

                acc = lax.fori_loop(
                    1, K1, add_j, gath_v[pl.ds(h * HE + c * L, L)])
                sums_v[pl.ds(h * HR + c * L, L)] = acc
                return carry

            lax.fori_loop(0, HR // L, reduce_chunk, 0)

        build_half(0)
        descs0 = fire_half(0)
        build_half(1)
        descs1 = fire_half(1)
        for d in descs0:
            d.wait()
        reduce_half(0)
        for d in descs1:
            d.wait()
        reduce_half(1)

        pltpu.sync_copy(sums_v, out_hbm.at[pl.ds(base, RPW)])

    return rowsum_kernel(data_phys, idx_t)


def _tc_softmax(a_raw):
    def body(x_ref, alpha_ref):
        x = x_ref[...]
        m = jnp.max(x)
        e = jnp.exp(x - m)
        alpha_ref[...] = e / jnp.sum(e)

    return pl.pallas_call(
        body,
        out_shape=jax.ShapeDtypeStruct((N,), jnp.float32),
    )(a_raw)


def kernel(data_input, neighbor_indices):
    idx = neighbor_indices[:, :K1].astype(jnp.int32)
    idx_t = idx.T.reshape(K1, N)
    data_phys = (data_input
                 .reshape(N // 8, 8, N // 128, 128)
                 .transpose(0, 2, 1, 3)
                 .reshape(N * N))
    a_raw = _sc_row_sums(data_phys, idx_t)
    alpha = _tc_softmax(a_raw)
    return (alpha, a_raw)


# speedup vs baseline: 1.0311x; 1.0275x over previous
import functools

import jax
import jax.numpy as jnp
from jax import lax
from jax.experimental import pallas as pl
from jax.experimental.pallas import tpu as pltpu
from jax.experimental.pallas import tpu_sc as plsc

N = 8192
K1 = 17
NC, NS, L = 1, 16, 16
NW = NC * NS
RPW = N // NW
HR = RPW // 2
HE = K1 * HR
E = K1 * RPW
CH = 128


def _sc_row_sums(data_phys, idx_t):
    mesh = plsc.VectorSubcoreMesh(
        core_axis_name="c", subcore_axis_name="s",
        num_cores=NC, num_subcores=NS)

    @functools.partial(
        pl.kernel,
        out_type=jax.ShapeDtypeStruct((N,), jnp.float32),
        mesh=mesh,
        compiler_params=pltpu.CompilerParams(needs_layout_passes=False),
        scratch_types=[
            pltpu.VMEM((K1, RPW), jnp.int32),
            pltpu.VMEM((RPW,), jnp.int32),
            pltpu.VMEM((E,), jnp.int32),
            pltpu.VMEM((E,), jnp.float32),
            pltpu.VMEM((RPW,), jnp.float32),
            pltpu.SemaphoreType.DMA,
            pltpu.SemaphoreType.DMA,
        ],
    )
    def rowsum_kernel(data_hbm, idx_hbm, out_hbm, cols_v, ioff_v, flat_v,
                      gath_v, sums_v, sem, csem):
        wid = lax.axis_index("s") * NC + lax.axis_index("c")
        base = wid * RPW

        cols_dma = pltpu.async_copy(
            idx_hbm.at[:, pl.ds(base, RPW)], cols_v, csem)

        def pre_chunk(c, carry):
            i = base + c * L + lax.iota(jnp.int32, L)
            ioff_v[pl.ds(c * L, L)] = ((i >> 3) << 16) + ((i & 7) << 7)
            return carry

        lax.fori_loop(0, RPW // L, pre_chunk, 0)
        cols_dma.wait()

        def build_half(h):
            def build_chunk(c, carry):
                r = h * HR + c * L
                ioff = ioff_v[pl.ds(r, L)]

                def build_j(j, carry):
                    col = cols_v[j, pl.ds(r, L)]
                    flat_v[pl.ds(h * HE + j * HR + c * L, L)] = (
                        ioff + ((col >> 7) << 10) + (col & 127))
                    return carry

                return lax.fori_loop(0, K1, build_j, carry)

            lax.fori_loop(0, HR // L, build_chunk, 0)

        def fire_half(h):
            return [
                pltpu.async_copy(
                    data_hbm.at[flat_v.at[pl.ds(h * HE + j * CH, CH)]],
                    gath_v.at[pl.ds(h * HE + j * CH, CH)], sem)
                for j in range(K1)
            ]

        def reduce_half(h):
            def reduce_chunk(c, carry):
                def add_j(j, acc):
                    return acc + gath_v[pl.ds(h * HE + j * HR + c * L, L)]

                acc = lax.fori_loop(
                    1, K1, add_j, gath_v[pl.ds(h * HE + c * L, L)])
                sums_v[pl.ds(h * HR + c * L, L)] = acc
                return carry

            lax.fori_loop(0, HR // L, reduce_chunk, 0)

        build_half(0)
        descs0 = fire_half(0)
        build_half(1)
        descs1 = fire_half(1)
        for d in descs0:
            d.wait()
        reduce_half(0)
        for d in descs1:
            d.wait()
        reduce_half(1)

        pltpu.sync_copy(sums_v, out_hbm.at[pl.ds(base, RPW)])

    return rowsum_kernel(data_phys, idx_t)


def _tc_softmax(a_raw):
    def body(x_ref, alpha_ref):
        x = x_ref[...]
        m = jnp.max(x)
        e = jnp.exp(x - m)
        alpha_ref[...] = e / jnp.sum(e)

    return pl.pallas_call(
        body,
        out_shape=jax.ShapeDtypeStruct((N,), jnp.float32),
    )(a_raw)


def kernel(data_input, neighbor_indices):
    idx = neighbor_indices[:, :K1].astype(jnp.int32)
    idx_t = idx.T.reshape(K1, N)
    data_phys = (data_input
                 .reshape(N // 8, 8, N // 128, 128)
                 .transpose(0, 2, 1, 3)
                 .reshape(N * N))
    a_raw = _sc_row_sums(data_phys, idx_t)
    alpha = _tc_softmax(a_raw)
    return (alpha, a_raw)
